# baseline (speedup 1.0000x reference)
import functools

import jax
import jax.numpy as jnp
from jax import lax
from jax.experimental import pallas as pl
from jax.experimental.pallas import tpu as pltpu
from jax.experimental.pallas import tpu_sc as plsc


def _emb_lookup(idx_grouped, weight, *, nw, n_ch, ch, d):
    n = nw * n_ch * ch
    per_w = n_ch * ch
    mesh = plsc.VectorSubcoreMesh(core_axis_name="c", subcore_axis_name="s")
    nbuf = 2

    @functools.partial(
        pl.kernel,
        out_type=jax.ShapeDtypeStruct((n, d), jnp.float32),
        mesh=mesh,
        scratch_types=[
            pltpu.VMEM((n_ch, ch), jnp.int32),
            pltpu.VMEM((nbuf, ch, d), jnp.float32),
            pltpu.SemaphoreType.DMA,
            pltpu.SemaphoreType.DMA,
        ],
        compiler_params=pltpu.CompilerParams(use_tc_tiling_on_sc=False),
    )
    def emb(idx_hbm, w_hbm, out_hbm, idx_v, rows_v, sem0, sem1):
        sems = (sem0, sem1)
        wid = lax.axis_index("s") * mesh.num_cores + lax.axis_index("c")
        base = wid * per_w
        pltpu.sync_copy(idx_hbm.at[wid], idx_v)
        for b in range(nbuf):
            pltpu.async_copy(w_hbm.at[idx_v.at[b]], rows_v.at[b], sems[b])

        @pl.loop(0, n_ch // nbuf)
        def _(g):
            j0 = g * nbuf
            for b in range(nbuf):
                j = j0 + b
                pltpu.make_async_copy(
                    w_hbm.at[idx_v.at[b]], rows_v.at[b], sems[b]
                ).wait()
                pltpu.sync_copy(
                    rows_v.at[b], out_hbm.at[pl.ds(base + j * ch, ch)]
                )
                nj = j + nbuf

                @pl.when(nj < n_ch)
                def _():
                    pltpu.async_copy(
                        w_hbm.at[idx_v.at[nj]], rows_v.at[b], sems[b]
                    )

    return emb(idx_grouped, weight)


def kernel(idx, weight):
    b, t = idx.shape
    v, d = weight.shape
    n = b * t
    nw = 32
    ch = 128
    per_w = n // nw
    n_ch = per_w // ch
    assert per_w * nw == n and n_ch * ch == per_w

    idx_grouped = idx.reshape(nw, n_ch, ch).astype(jnp.int32)
    rows = _emb_lookup(idx_grouped, weight, nw=nw, n_ch=n_ch, ch=ch, d=d)
    return rows.reshape(b, t, d)


# speedup vs baseline: 1.3278x; 1.3278x over previous
import functools

import jax
import jax.numpy as jnp
from jax import lax
from jax.experimental import pallas as pl
from jax.experimental.pallas import tpu as pltpu
from jax.experimental.pallas import tpu_sc as plsc

_LANES = 128


def _emb_lookup(idx_grouped, weight, *, nw, n_ch, ch, d):
    n = nw * n_ch * ch
    per_w = n_ch * ch
    mesh = plsc.VectorSubcoreMesh(core_axis_name="c", subcore_axis_name="s")
    nbuf = 2

    @functools.partial(
        pl.kernel,
        out_type=jax.ShapeDtypeStruct((n, _LANES), jnp.float32),
        mesh=mesh,
        scratch_types=[
            pltpu.VMEM((n_ch, ch), jnp.int32),
            pltpu.VMEM((nbuf, ch, d), jnp.float32),
            pltpu.SemaphoreType.DMA,
            pltpu.SemaphoreType.DMA,
        ],
        compiler_params=pltpu.CompilerParams(use_tc_tiling_on_sc=False),
    )
    def emb(idx_hbm, w_hbm, out_hbm, idx_v, rows_v, sem0, sem1):
        sems = (sem0, sem1)
        wid = lax.axis_index("s") * mesh.num_cores + lax.axis_index("c")
        base = wid * per_w
        pltpu.sync_copy(idx_hbm.at[wid], idx_v)
        for b in range(nbuf):
            pltpu.async_copy(w_hbm.at[idx_v.at[b]], rows_v.at[b], sems[b])

        @pl.loop(0, n_ch // nbuf)
        def _(g):
            j0 = g * nbuf
            for b in range(nbuf):
                j = j0 + b
                pltpu.make_async_copy(
                    w_hbm.at[idx_v.at[b]], rows_v.at[b], sems[b]
                ).wait()
                pltpu.sync_copy(
                    rows_v.at[b],
                    out_hbm.at[pl.ds(base + j * ch, ch), pl.ds(0, d)],
                )
                nj = j + nbuf

                @pl.when(nj < n_ch)
                def _():
                    pltpu.async_copy(
                        w_hbm.at[idx_v.at[nj]], rows_v.at[b], sems[b]
                    )

    return emb(idx_grouped, weight)


def kernel(idx, weight):
    b, t = idx.shape
    v, d = weight.shape
    n = b * t
    nw = 32
    ch = 128
    per_w = n // nw
    n_ch = per_w // ch
    assert per_w * nw == n and n_ch * ch == per_w

    idx_grouped = idx.reshape(nw, n_ch, ch).astype(jnp.int32)
    rows = _emb_lookup(idx_grouped, weight, nw=nw, n_ch=n_ch, ch=ch, d=d)
    return rows[:, :d].reshape(b, t, d)


# speedup vs baseline: 1.3285x; 1.0005x over previous
import functools

import jax
import jax.numpy as jnp
from jax import lax
from jax.experimental import pallas as pl
from jax.experimental.pallas import tpu as pltpu
from jax.experimental.pallas import tpu_sc as plsc

_LANES = 128


def _emb_lookup(idx_grouped, weight, *, nw, n_ch, ch, d):
    n = nw * n_ch * ch
    per_w = n_ch * ch
    mesh = plsc.VectorSubcoreMesh(core_axis_name="c", subcore_axis_name="s")
    nbuf = 2

    @functools.partial(
        pl.kernel,
        out_type=jax.ShapeDtypeStruct((n, _LANES), jnp.float32),
        mesh=mesh,
        scratch_types=[
            pltpu.VMEM((n_ch, ch), jnp.int32),
            pltpu.VMEM((nbuf, ch, d), jnp.float32),
            pltpu.SemaphoreType.DMA,
            pltpu.SemaphoreType.DMA,
        ],
        compiler_params=pltpu.CompilerParams(
            use_tc_tiling_on_sc=False, skip_device_barrier=True
        ),
    )
    def emb(idx_hbm, w_hbm, out_hbm, idx_v, rows_v, sem0, sem1):
        sems = (sem0, sem1)
        wid = lax.axis_index("s") * mesh.num_cores + lax.axis_index("c")
        base = wid * per_w
        pltpu.sync_copy(idx_hbm.at[wid], idx_v)
        for b in range(nbuf):
            pltpu.async_copy(w_hbm.at[idx_v.at[b]], rows_v.at[b], sems[b])

        @pl.loop(0, n_ch // nbuf)
        def _(g):
            j0 = g * nbuf
            for b in range(nbuf):
                j = j0 + b
                pltpu.make_async_copy(
                    w_hbm.at[idx_v.at[b]], rows_v.at[b], sems[b]
                ).wait()
                pltpu.sync_copy(
                    rows_v.at[b],
                    out_hbm.at[pl.ds(base + j * ch, ch), pl.ds(0, d)],
                )
                nj = j + nbuf

                @pl.when(nj < n_ch)
                def _():
                    pltpu.async_copy(
                        w_hbm.at[idx_v.at[nj]], rows_v.at[b], sems[b]
                    )

    return emb(idx_grouped, weight)


def kernel(idx, weight):
    b, t = idx.shape
    v, d = weight.shape
    n = b * t
    nw = 32
    ch = 128
    per_w = n // nw
    n_ch = per_w // ch
    assert per_w * nw == n and n_ch * ch == per_w

    idx_grouped = idx.reshape(nw, n_ch, ch).astype(jnp.int32)
    rows = _emb_lookup(idx_grouped, weight, nw=nw, n_ch=n_ch, ch=ch, d=d)
    return rows[:, :d].reshape(b, t, d)
